# Initial kernel scaffold; baseline (speedup 1.0000x reference)
import jax
import jax.numpy as jnp
from jax import lax
from jax.experimental import pallas as pl
from jax.experimental.pallas import tpu as pltpu
from jax.experimental.pallas import tpu_sc as plsc

N = 10000
E = 320000
D = 128
H = 128
A = 64
HP = 144
NC, NS, L = 2, 16, 16
NW = NC * NS
EPW = E // NW
CH = 80
NCHUNK = EPW // CH
RPT = N // NS


def _pre_body(x_ref, w_ref, asrc_ref, adst_ref, ht_ref, ssrc_ref, sdst_ref):
    h = jnp.dot(x_ref[...], w_ref[...], preferred_element_type=jnp.float32)
    blk = h.shape[0]
    ones = jnp.ones((blk, 1), jnp.float32)
    zeros = jnp.zeros((blk, HP - D - 1), jnp.float32)
    ht_ref[...] = jnp.concatenate([h, ones, zeros], axis=1)
    ssrc_ref[...] = jnp.sum(h * asrc_ref[...], axis=1, keepdims=True)
    sdst_ref[...] = jnp.sum(h * adst_ref[...], axis=1, keepdims=True)


def _pre(x, W_gat, a_src, a_dst):
    blk = 2000
    grid = (N // blk,)
    return pl.pallas_call(
        _pre_body,
        grid=grid,
        in_specs=[
            pl.BlockSpec((blk, D), lambda i: (i, 0)),
            pl.BlockSpec((D, H), lambda i: (0, 0)),
            pl.BlockSpec((1, H), lambda i: (0, 0)),
            pl.BlockSpec((1, H), lambda i: (0, 0)),
        ],
        out_specs=[
            pl.BlockSpec((blk, HP), lambda i: (i, 0)),
            pl.BlockSpec((blk, 1), lambda i: (i, 0)),
            pl.BlockSpec((blk, 1), lambda i: (i, 0)),
        ],
        out_shape=[
            jax.ShapeDtypeStruct((N, HP), jnp.float32),
            jax.ShapeDtypeStruct((N, 1), jnp.float32),
            jax.ShapeDtypeStruct((N, 1), jnp.float32),
        ],
    )(x, W_gat, a_src.reshape(1, H), a_dst.reshape(1, H))


def _sc_body(ht_hbm, ssrc_hbm, sdst_hbm, src_hbm, dst_hbm, zeros_hbm,
             out0_hbm, out1_hbm,
             ssrc_v, sdst_v, srcidx_v, dstidx_v, w_v, rows_v, acc_sh, gsem):
    cid = lax.axis_index("c")
    sid = lax.axis_index("s")
    wid = sid * NC + cid

    pltpu.sync_copy(ssrc_hbm, ssrc_v)
    pltpu.sync_copy(sdst_hbm, sdst_v)
    pltpu.sync_copy(src_hbm.at[wid], srcidx_v)
    pltpu.sync_copy(dst_hbm.at[wid], dstidx_v)

    @pl.loop(0, NCHUNK)
    def _(i):
        for j in range(CH // L):
            sl = pl.ds(j * L, L)
            si = plsc.load_gather(ssrc_v, [srcidx_v[i, sl]])
            di = plsc.load_gather(sdst_v, [dstidx_v[i, sl]])
            e = si + di
            e = jnp.maximum(e, 0.2 * e)
            w_v[pl.ds(i * CH + j * L, L)] = jnp.exp(e)

    pltpu.sync_copy(zeros_hbm.at[pl.ds(sid * RPT, RPT)],
                    acc_sh.at[pl.ds(sid * RPT, RPT)])
    plsc.subcore_barrier()

    @pl.loop(0, NCHUNK)
    def _(i):
        pltpu.async_copy(ht_hbm.at[srcidx_v.at[i]], rows_v, gsem).wait()

        @pl.loop(0, CH)
        def _(r):
            wb = plsc.load_gather(w_v, [jnp.full((L,), i * CH + r, jnp.int32)])
            for k in range(HP // L):
                sl = pl.ds(k * L, L)
                rows_v[r, sl] = rows_v[r, sl] * wb

        pltpu.sync_copy(rows_v, acc_sh.at[dstidx_v.at[i]], add=True)

    plsc.subcore_barrier()

    row_sl = pl.ds(sid * RPT, RPT)

    @pl.when(cid == 0)
    def _():
        pltpu.sync_copy(acc_sh.at[row_sl], out0_hbm.at[row_sl])

    @pl.when(cid == 1)
    def _():
        pltpu.sync_copy(acc_sh.at[row_sl], out1_hbm.at[row_sl])


def _sc_aggregate(ht, ssrc, sdst, src3, dst3, zeros):
    mesh = plsc.VectorSubcoreMesh(core_axis_name="c", subcore_axis_name="s")
    kern = pl.kernel(
        _sc_body,
        out_type=[
            jax.ShapeDtypeStruct((N, HP), jnp.float32),
            jax.ShapeDtypeStruct((N, HP), jnp.float32),
        ],
        mesh=mesh,
        scratch_types=[
            pltpu.VMEM((N,), jnp.float32),
            pltpu.VMEM((N,), jnp.float32),
            pltpu.VMEM((NCHUNK, CH), jnp.int32),
            pltpu.VMEM((NCHUNK, CH), jnp.int32),
            pltpu.VMEM((EPW,), jnp.float32),
            pltpu.VMEM((CH, HP), jnp.float32),
            pltpu.VMEM_SHARED((N, HP), jnp.float32),
            pltpu.SemaphoreType.DMA,
        ],
    )
    return kern(ht, ssrc, sdst, src3, dst3, zeros)


def _post_body(acc0_ref, acc1_ref, bn0g_ref, bn0b_ref, fc1w_ref, fc1b_ref,
               fc2w_ref, fc2b_ref, bn2g_ref, bn2b_ref, fc3w_ref, fc3b_ref,
               out_ref):
    unnorm = acc0_ref[:, :D] + acc1_ref[:, :D]
    den = acc0_ref[:, D:D + 1] + acc1_ref[:, D:D + 1]
    h0 = jnp.maximum(unnorm / (den + 1e-16), 0.0)

    m0 = jnp.mean(h0, axis=0, keepdims=True)
    v0 = jnp.mean((h0 - m0) ** 2, axis=0, keepdims=True)
    h0n = (h0 - m0) / jnp.sqrt(v0 + 1e-5) * bn0g_ref[...] + bn0b_ref[...]

    h1 = lax.dot_general(h0n, fc1w_ref[...], (((1,), (1,)), ((), ())),
                         preferred_element_type=jnp.float32) + fc1b_ref[...]
    h1 = jnp.maximum(h1, 0.0)
    h2 = lax.dot_general(h1, fc2w_ref[...], (((1,), (1,)), ((), ())),
                         preferred_element_type=jnp.float32) + fc2b_ref[...]
    h2 = jnp.maximum(h2, 0.0)

    m2 = jnp.mean(h2, axis=0, keepdims=True)
    v2 = jnp.mean((h2 - m2) ** 2, axis=0, keepdims=True)
    h2n = (h2 - m2) / jnp.sqrt(v2 + 1e-5) * bn2g_ref[...] + bn2b_ref[...]

    act = lax.dot_general(h2n, fc3w_ref[...], (((1,), (1,)), ((), ())),
                          preferred_element_type=jnp.float32) + fc3b_ref[...]
    amax = jnp.max(act, axis=1, keepdims=True)
    ex = jnp.exp(act - amax)
    out_ref[...] = ex / jnp.sum(ex, axis=1, keepdims=True)


def _post(acc0, acc1, bn0_gamma, bn0_beta, fc1_w, fc1_b, fc2_w, fc2_b,
          bn2_gamma, bn2_beta, fc3_w, fc3_b):
    return pl.pallas_call(
        _post_body,
        out_shape=jax.ShapeDtypeStruct((N, A), jnp.float32),
    )(acc0, acc1,
      bn0_gamma.reshape(1, H), bn0_beta.reshape(1, H),
      fc1_w, fc1_b.reshape(1, H),
      fc2_w, fc2_b.reshape(1, H),
      bn2_gamma.reshape(1, H), bn2_beta.reshape(1, H),
      fc3_w, fc3_b.reshape(1, A))


def kernel(x, W_gat, a_src, a_dst, bn0_gamma, bn0_beta, fc1_w, fc1_b,
           fc2_w, fc2_b, bn2_gamma, bn2_beta, fc3_w, fc3_b, edge_index):
    ht, ssrc, sdst = _pre(x, W_gat, a_src, a_dst)
    src3 = edge_index[0].reshape(NW, NCHUNK, CH)
    dst3 = edge_index[1].reshape(NW, NCHUNK, CH)
    zeros = jnp.zeros((N, HP), jnp.float32)
    acc0, acc1 = _sc_aggregate(ht, ssrc.reshape(N), sdst.reshape(N),
                               src3, dst3, zeros)
    return _post(acc0, acc1, bn0_gamma, bn0_beta, fc1_w, fc1_b,
                 fc2_w, fc2_b, bn2_gamma, bn2_beta, fc3_w, fc3_b)


# speedup vs baseline: 28.2394x; 28.2394x over previous
import dataclasses

import jax
import jax.numpy as jnp
from jax import lax
from jax.experimental import pallas as pl
from jax.experimental.pallas import tpu as pltpu
from jax.experimental.pallas import tpu_sc as plsc

N = 10000
E = 320000
D = 128
H = 128
A = 64
HP = 144
NC, NS, L = 2, 16, 16
NW = NC * NS
EPW = E // NW
CH = 80
NCHUNK = EPW // CH
RPT = 624
RREM = N - NS * RPT


def _pre_body(x_ref, w_ref, asrc_ref, adst_ref, ht_ref, ssrc_ref, sdst_ref):
    h = jnp.dot(x_ref[...], w_ref[...], preferred_element_type=jnp.float32)
    blk = h.shape[0]
    ones = jnp.ones((blk, 1), jnp.float32)
    zeros = jnp.zeros((blk, HP - D - 1), jnp.float32)
    ht_ref[...] = jnp.concatenate([h, ones, zeros], axis=1)
    ssrc_ref[...] = jnp.sum(h * asrc_ref[...], axis=1, keepdims=True)
    sdst_ref[...] = jnp.sum(h * adst_ref[...], axis=1, keepdims=True)


def _pre(x, W_gat, a_src, a_dst):
    blk = 2000
    grid = (N // blk,)
    return pl.pallas_call(
        _pre_body,
        grid=grid,
        in_specs=[
            pl.BlockSpec((blk, D), lambda i: (i, 0)),
            pl.BlockSpec((D, H), lambda i: (0, 0)),
            pl.BlockSpec((1, H), lambda i: (0, 0)),
            pl.BlockSpec((1, H), lambda i: (0, 0)),
        ],
        out_specs=[
            pl.BlockSpec((blk, HP), lambda i: (i, 0)),
            pl.BlockSpec((blk, 1), lambda i: (i, 0)),
            pl.BlockSpec((blk, 1), lambda i: (i, 0)),
        ],
        out_shape=[
            jax.ShapeDtypeStruct((N, HP), jnp.float32),
            jax.ShapeDtypeStruct((N, 1), jnp.float32),
            jax.ShapeDtypeStruct((N, 1), jnp.float32),
        ],
    )(x, W_gat, a_src.reshape(1, H), a_dst.reshape(1, H))


def _sc_params():
    cp = pltpu.CompilerParams()
    fields = pltpu.CompilerParams.__dataclass_fields__
    if "needs_layout_passes" in fields:
        cp = dataclasses.replace(cp, needs_layout_passes=False)
    if "use_tc_tiling_on_sc" in fields:
        cp = dataclasses.replace(cp, use_tc_tiling_on_sc=False)
    return cp


def _mod_nchunk(i):
    return jnp.where(i >= NCHUNK, i - NCHUNK, i)


def _scw_body(ssrc_hbm, sdst_hbm, src_hbm, dst_hbm, w_hbm,
              ssrc_t, sdst_t, sidx, didx, wall, isem):
    cid = lax.axis_index("c")
    sid = lax.axis_index("s")
    wid = sid * NC + cid

    c1 = pltpu.async_copy(ssrc_hbm, ssrc_t, isem)
    c2 = pltpu.async_copy(sdst_hbm, sdst_t, isem)
    c3 = pltpu.async_copy(src_hbm.at[wid], sidx, isem)
    c4 = pltpu.async_copy(dst_hbm.at[wid], didx, isem)
    c1.wait()
    c2.wait()
    c3.wait()
    c4.wait()

    @plsc.parallel_loop(0, EPW // L, unroll=4)
    def _(g):
        sl = pl.ds(g * L, L)
        si = plsc.load_gather(ssrc_t, [sidx[sl]])
        di = plsc.load_gather(sdst_t, [didx[sl]])
        e = si + di
        e = jnp.maximum(e, 0.2 * e)
        wall[sl] = jnp.exp(e)

    pltpu.sync_copy(wall, w_hbm.at[wid])


def _sc_weights(ssrc, sdst, src2, dst2):
    kern = pl.kernel(
        _scw_body,
        out_type=[jax.ShapeDtypeStruct((NW, EPW), jnp.float32)],
        mesh=plsc.VectorSubcoreMesh(core_axis_name="c", subcore_axis_name="s"),
        scratch_types=[
            pltpu.VMEM((N,), jnp.float32),
            pltpu.VMEM((N,), jnp.float32),
            pltpu.VMEM((EPW,), jnp.int32),
            pltpu.VMEM((EPW,), jnp.int32),
            pltpu.VMEM((EPW,), jnp.float32),
            pltpu.SemaphoreType.DMA,
        ],
        compiler_params=_sc_params(),
    )
    return kern(ssrc, sdst, src2, dst2)[0]


def _scagg_body(ht_hbm, src_hbm, dst_hbm, w_hbm, zeros_hbm,
                out0_hbm, out1_hbm,
                rows, sidx, didx, wch, acc_sh, gsem0, gsem1,
                ssem0, ssem1, isem):
    cid = lax.axis_index("c")
    sid = lax.axis_index("s")
    wid = sid * NC + cid

    zoff = pl.multiple_of(sid * RPT, 8)
    pltpu.sync_copy(zeros_hbm.at[pl.ds(zoff, RPT)],
                    acc_sh.at[pl.ds(zoff, RPT)])

    @pl.when(sid == NS - 1)
    def _():
        pltpu.sync_copy(zeros_hbm.at[pl.ds(NS * RPT, RREM)],
                        acc_sh.at[pl.ds(NS * RPT, RREM)])

    plsc.subcore_barrier()

    pltpu.sync_copy(src_hbm.at[wid, 0], sidx.at[0])
    pltpu.sync_copy(dst_hbm.at[wid, 0], didx.at[0])
    pltpu.sync_copy(w_hbm.at[wid, 0], wch.at[0])
    pltpu.async_copy(ht_hbm.at[sidx.at[0]], rows.at[0], gsem0)
    pltpu.async_copy(src_hbm.at[wid, 1], sidx.at[1], isem)
    pltpu.async_copy(dst_hbm.at[wid, 1], didx.at[1], isem)
    pltpu.async_copy(w_hbm.at[wid, 1], wch.at[1], isem)

    def scale(b, q):
        @plsc.parallel_loop(0, CH, unroll=4)
        def _(r):
            wb = plsc.load_gather(wch.at[q], [jnp.full((L,), r, jnp.int32)])
            for k in range(HP // L):
                sl = pl.ds(k * L, L)
                rows[b, r, sl] = rows[b, r, sl] * wb

    def wait_idx(q):
        pltpu.make_async_copy(src_hbm.at[wid, 0], sidx.at[q], isem).wait()
        pltpu.make_async_copy(dst_hbm.at[wid, 0], didx.at[q], isem).wait()
        pltpu.make_async_copy(w_hbm.at[wid, 0], wch.at[q], isem).wait()

    gsems = (gsem0, gsem1)
    ssems = (ssem0, ssem1)

    def substep(j, s, first, last):
        b, nb = s % 2, (s + 1) % 2
        q1, q2 = (s + 1) % 4, (s + 2) % 4
        if not last:
            wait_idx(q1)

            def _wait_prev_scatter():
                pltpu.make_async_copy(rows.at[nb], acc_sh.at[didx.at[q1]],
                                      ssems[nb]).wait()

            if first:
                pl.when(j >= 1)(_wait_prev_scatter)
            else:
                _wait_prev_scatter()
            pltpu.async_copy(ht_hbm.at[sidx.at[q1]], rows.at[nb], gsems[nb])
        pltpu.make_async_copy(ht_hbm.at[sidx.at[s % 4]], rows.at[b],
                              gsems[b]).wait()
        scale(b, s % 4)
        if last:
            pltpu.sync_copy(rows.at[b], acc_sh.at[didx.at[s % 4]], add=True)
        else:
            pltpu.async_copy(rows.at[b], acc_sh.at[didx.at[s % 4]],
                             ssems[b], add=True)
            j2 = _mod_nchunk(j + 2)
            pltpu.async_copy(src_hbm.at[wid, j2], sidx.at[q2], isem)
            pltpu.async_copy(dst_hbm.at[wid, j2], didx.at[q2], isem)
            pltpu.async_copy(w_hbm.at[wid, j2], wch.at[q2], isem)

    @pl.loop(0, NCHUNK - 1, step=4)
    def _(i):
        substep(i, 0, True, False)
        substep(i + 1, 1, False, False)
        substep(i + 2, 2, False, False)
        substep(i + 3, 3, False, False)

    substep(NCHUNK - 1, 0, False, True)
    pltpu.make_async_copy(rows.at[1], acc_sh.at[didx.at[1]], ssem1).wait()
    wait_idx(1)

    plsc.subcore_barrier()

    row_sl = pl.ds(pl.multiple_of(sid * RPT, 8), RPT)
    rem_sl = pl.ds(NS * RPT, RREM)

    @pl.when(cid == 0)
    def _():
        pltpu.sync_copy(acc_sh.at[row_sl], out0_hbm.at[row_sl])

        @pl.when(sid == NS - 1)
        def _():
            pltpu.sync_copy(acc_sh.at[rem_sl], out0_hbm.at[rem_sl])

    @pl.when(cid == 1)
    def _():
        pltpu.sync_copy(acc_sh.at[row_sl], out1_hbm.at[row_sl])

        @pl.when(sid == NS - 1)
        def _():
            pltpu.sync_copy(acc_sh.at[rem_sl], out1_hbm.at[rem_sl])


def _sc_aggregate(ht, ssrc, sdst, src3, dst3, zeros):
    w3 = _sc_weights(ssrc, sdst, src3.reshape(NW, EPW),
                     dst3.reshape(NW, EPW)).reshape(NW, NCHUNK, CH)
    kern = pl.kernel(
        _scagg_body,
        out_type=[
            jax.ShapeDtypeStruct((N, HP), jnp.float32),
            jax.ShapeDtypeStruct((N, HP), jnp.float32),
        ],
        mesh=plsc.VectorSubcoreMesh(core_axis_name="c", subcore_axis_name="s"),
        scratch_types=[
            pltpu.VMEM((2, CH, HP), jnp.float32),
            pltpu.VMEM((4, CH), jnp.int32),
            pltpu.VMEM((4, CH), jnp.int32),
            pltpu.VMEM((4, CH), jnp.float32),
            pltpu.VMEM_SHARED((N, HP), jnp.float32),
            pltpu.SemaphoreType.DMA,
            pltpu.SemaphoreType.DMA,
            pltpu.SemaphoreType.DMA,
            pltpu.SemaphoreType.DMA,
            pltpu.SemaphoreType.DMA,
        ],
        compiler_params=_sc_params(),
    )
    return kern(ht, src3, dst3, w3, zeros)


def _post_body(acc0_ref, acc1_ref, bn0g_ref, bn0b_ref, fc1w_ref, fc1b_ref,
               fc2w_ref, fc2b_ref, bn2g_ref, bn2b_ref, fc3w_ref, fc3b_ref,
               out_ref):
    unnorm = acc0_ref[:, :D] + acc1_ref[:, :D]
    den = acc0_ref[:, D:D + 1] + acc1_ref[:, D:D + 1]
    h0 = jnp.maximum(unnorm / (den + 1e-16), 0.0)

    m0 = jnp.mean(h0, axis=0, keepdims=True)
    v0 = jnp.mean((h0 - m0) ** 2, axis=0, keepdims=True)
    h0n = (h0 - m0) / jnp.sqrt(v0 + 1e-5) * bn0g_ref[...] + bn0b_ref[...]

    h1 = lax.dot_general(h0n, fc1w_ref[...], (((1,), (1,)), ((), ())),
                         preferred_element_type=jnp.float32) + fc1b_ref[...]
    h1 = jnp.maximum(h1, 0.0)
    h2 = lax.dot_general(h1, fc2w_ref[...], (((1,), (1,)), ((), ())),
                         preferred_element_type=jnp.float32) + fc2b_ref[...]
    h2 = jnp.maximum(h2, 0.0)

    m2 = jnp.mean(h2, axis=0, keepdims=True)
    v2 = jnp.mean((h2 - m2) ** 2, axis=0, keepdims=True)
    h2n = (h2 - m2) / jnp.sqrt(v2 + 1e-5) * bn2g_ref[...] + bn2b_ref[...]

    act = lax.dot_general(h2n, fc3w_ref[...], (((1,), (1,)), ((), ())),
                          preferred_element_type=jnp.float32) + fc3b_ref[...]
    amax = jnp.max(act, axis=1, keepdims=True)
    ex = jnp.exp(act - amax)
    out_ref[...] = ex / jnp.sum(ex, axis=1, keepdims=True)


def _post(acc0, acc1, bn0_gamma, bn0_beta, fc1_w, fc1_b, fc2_w, fc2_b,
          bn2_gamma, bn2_beta, fc3_w, fc3_b):
    return pl.pallas_call(
        _post_body,
        out_shape=jax.ShapeDtypeStruct((N, A), jnp.float32),
    )(acc0, acc1,
      bn0_gamma.reshape(1, H), bn0_beta.reshape(1, H),
      fc1_w, fc1_b.reshape(1, H),
      fc2_w, fc2_b.reshape(1, H),
      bn2_gamma.reshape(1, H), bn2_beta.reshape(1, H),
      fc3_w, fc3_b.reshape(1, A))


def kernel(x, W_gat, a_src, a_dst, bn0_gamma, bn0_beta, fc1_w, fc1_b,
           fc2_w, fc2_b, bn2_gamma, bn2_beta, fc3_w, fc3_b, edge_index):
    ht, ssrc, sdst = _pre(x, W_gat, a_src, a_dst)
    src3 = edge_index[0].reshape(NW, NCHUNK, CH)
    dst3 = edge_index[1].reshape(NW, NCHUNK, CH)
    zeros = jnp.zeros((N, HP), jnp.float32)
    acc0, acc1 = _sc_aggregate(ht, ssrc.reshape(N), sdst.reshape(N),
                               src3, dst3, zeros)
    return _post(acc0, acc1, bn0_gamma, bn0_beta, fc1_w, fc1_b,
                 fc2_w, fc2_b, bn2_gamma, bn2_beta, fc3_w, fc3_b)
